# Initial kernel scaffold; baseline (speedup 1.0000x reference)
import jax
import jax.numpy as jnp
from jax.experimental import pallas as pl


def kernel(x, node_attr, edge_index, edge_attr, edge_len_emb, W_sc, W_lin1, W_lin2, W_alpha, W_mlp1, W_mlp2):
    raise NotImplementedError("write your pallas kernel here")


# speedup vs baseline: 1.7362x; 1.7362x over previous
import functools
import numpy as np
import jax
import jax.numpy as jnp
from jax import lax
from jax.experimental import pallas as pl
from jax.experimental.pallas import tpu as pltpu
from jax.experimental.pallas import tpu_sc as plsc

_N = 10000
_E = 320000
_D = 128
_A = 8
_R = 16
_H = 64
_NUM_NEIGHBORS = 32.0

_NW = 32
_CHUNK = 128
_NCH = 79
_EPW = _CHUNK * _NCH
_EPAD = _EPW * _NW
_NPAD = 10240
_NODES_PER_TILE = _NPAD // 16


def _node_fctp_body(x_ref, a_ref, wsc_ref, wl1_ref, nsc_ref, nf_ref):
    x = x_ref[...]
    a = a_ref[...]
    acc_sc = jnp.zeros_like(nsc_ref)
    acc_l1 = jnp.zeros_like(nf_ref)
    for j in range(_A):
        aj = a[:, j:j + 1]
        acc_sc += aj * jnp.dot(x, wsc_ref[j], preferred_element_type=jnp.float32)
        acc_l1 += aj * jnp.dot(x, wl1_ref[j], preferred_element_type=jnp.float32)
    scale = 1.0 / np.sqrt(_D * _A)
    nsc_ref[...] = acc_sc * scale
    nf_ref[...] = acc_l1 * scale


def _edge_mlp_body(emb_ref, ea_ref, w1_ref, w2_ref, ew_ref):
    h = jnp.dot(emb_ref[...], w1_ref[...], preferred_element_type=jnp.float32)
    h = jax.nn.silu(h * (1.0 / np.sqrt(_R)))
    w = jnp.dot(h, w2_ref[...], preferred_element_type=jnp.float32)
    ew_ref[...] = w * (1.0 / np.sqrt(_H)) * ea_ref[...]


def _sc_edge_body(nf_hbm, ew_hbm, isrc_hbm, jdst_hbm, zeros_hbm, out_hbm,
                  iv, jv, rows, ewv, agg, sem):
    c = lax.axis_index("c")
    s = lax.axis_index("s")
    wid = c * 16 + s
    stripe = pl.ds(s * _NODES_PER_TILE, _NODES_PER_TILE)
    pltpu.sync_copy(zeros_hbm.at[stripe], agg.at[stripe])
    plsc.subcore_barrier()

    base = wid * _EPW

    @pl.loop(0, _NCH)
    def _chunk(k):
        off = pl.multiple_of(base + k * _CHUNK, _CHUNK)
        pltpu.sync_copy(isrc_hbm.at[pl.ds(off, _CHUNK)], iv)
        pltpu.sync_copy(jdst_hbm.at[pl.ds(off, _CHUNK)], jv)
        pltpu.async_copy(nf_hbm.at[iv], rows, sem).wait()
        pltpu.sync_copy(ew_hbm.at[pl.ds(off, _CHUNK)], ewv)

        @pl.loop(0, _CHUNK)
        def _row(e):
            for q in range(_D // 16):
                sl = pl.ds(q * 16, 16)
                rows[e, sl] = rows[e, sl] * ewv[e, sl]

        pltpu.sync_copy(rows, agg.at[jv], add=True)

    plsc.subcore_barrier()
    pltpu.sync_copy(agg.at[stripe], out_hbm.at[c, stripe])


def _sc_edge_pass(nf, ew, isrc_p, jdst_p):
    f32 = jnp.float32
    nf_p = jnp.pad(nf, ((0, _NPAD - _N), (0, 0)))
    zeros_nd = jnp.zeros((_NPAD, _D), f32)
    mesh = plsc.VectorSubcoreMesh(core_axis_name="c", subcore_axis_name="s",
                                  num_cores=2, num_subcores=16)
    return pl.kernel(
        _sc_edge_body,
        out_type=jax.ShapeDtypeStruct((2, _NPAD, _D), f32),
        mesh=mesh,
        scratch_types=[
            pltpu.VMEM((_CHUNK,), jnp.int32),
            pltpu.VMEM((_CHUNK,), jnp.int32),
            pltpu.VMEM((_CHUNK, _D), f32),
            pltpu.VMEM((_CHUNK, _D), f32),
            pltpu.VMEM_SHARED((_NPAD, _D), f32),
            pltpu.SemaphoreType.DMA,
        ],
    )(nf_p, ew, isrc_p, jdst_p, zeros_nd)


def _final_body(p_ref, a_ref, nsc_ref, wl2_ref, wa_ref, out_ref):
    agg = (p_ref[0] + p_ref[1]) * (1.0 / np.sqrt(_NUM_NEIGHBORS))
    a = a_ref[...]
    acc = jnp.zeros_like(out_ref)
    for j in range(_A):
        acc += a[:, j:j + 1] * jnp.dot(agg, wl2_ref[j],
                                       preferred_element_type=jnp.float32)
    scale = 1.0 / np.sqrt(_D * _A)
    nco = acc * scale
    va = jnp.dot(agg, wa_ref[...], preferred_element_type=jnp.float32)
    alpha = jnp.sum(va * a, axis=1, keepdims=True) * scale
    out_ref[...] = nsc_ref[...] + alpha * nco


def kernel(x, node_attr, edge_index, edge_attr, edge_len_emb,
           W_sc, W_lin1, W_lin2, W_alpha, W_mlp1, W_mlp2):
    f32 = jnp.float32
    x = x.astype(f32)
    isrc = edge_index[0].astype(jnp.int32)
    jdst = edge_index[1].astype(jnp.int32)

    wsc_t = jnp.transpose(W_sc, (1, 0, 2))
    wl1_t = jnp.transpose(W_lin1, (1, 0, 2))
    wl2_t = jnp.transpose(W_lin2, (1, 0, 2))

    bn = 1000
    grid_n = _N // bn
    nsc, nf = pl.pallas_call(
        _node_fctp_body,
        grid=(grid_n,),
        in_specs=[
            pl.BlockSpec((bn, _D), lambda i: (i, 0)),
            pl.BlockSpec((bn, _A), lambda i: (i, 0)),
            pl.BlockSpec((_A, _D, _D), lambda i: (0, 0, 0)),
            pl.BlockSpec((_A, _D, _D), lambda i: (0, 0, 0)),
        ],
        out_specs=[
            pl.BlockSpec((bn, _D), lambda i: (i, 0)),
            pl.BlockSpec((bn, _D), lambda i: (i, 0)),
        ],
        out_shape=[
            jax.ShapeDtypeStruct((_N, _D), f32),
            jax.ShapeDtypeStruct((_N, _D), f32),
        ],
    )(x, node_attr, wsc_t, wl1_t)

    emb_p = jnp.pad(edge_len_emb.astype(f32), ((0, _EPAD - _E), (0, 0)))
    ea_p = jnp.pad(edge_attr.astype(f32), ((0, _EPAD - _E), (0, 0)))
    isrc_p = jnp.pad(isrc, (0, _EPAD - _E))
    jdst_p = jnp.pad(jdst, (0, _EPAD - _E))

    be = 4096
    grid_e = _EPAD // be
    ew = pl.pallas_call(
        _edge_mlp_body,
        grid=(grid_e,),
        in_specs=[
            pl.BlockSpec((be, _R), lambda i: (i, 0)),
            pl.BlockSpec((be, 1), lambda i: (i, 0)),
            pl.BlockSpec((_R, _H), lambda i: (0, 0)),
            pl.BlockSpec((_H, _D), lambda i: (0, 0)),
        ],
        out_specs=pl.BlockSpec((be, _D), lambda i: (i, 0)),
        out_shape=jax.ShapeDtypeStruct((_EPAD, _D), f32),
    )(emb_p, ea_p, W_mlp1.astype(f32), W_mlp2.astype(f32))

    partials = _sc_edge_pass(nf, ew, isrc_p, jdst_p)

    wa_mat = W_alpha.reshape(_D, _A).astype(f32)
    out = pl.pallas_call(
        _final_body,
        grid=(grid_n,),
        in_specs=[
            pl.BlockSpec((2, bn, _D), lambda i: (0, i, 0)),
            pl.BlockSpec((bn, _A), lambda i: (i, 0)),
            pl.BlockSpec((bn, _D), lambda i: (i, 0)),
            pl.BlockSpec((_A, _D, _D), lambda i: (0, 0, 0)),
            pl.BlockSpec((_D, _A), lambda i: (0, 0)),
        ],
        out_specs=pl.BlockSpec((bn, _D), lambda i: (i, 0)),
        out_shape=jax.ShapeDtypeStruct((_N, _D), f32),
    )(partials, node_attr, nsc, wl2_t, wa_mat)
    return out


# speedup vs baseline: 2.3096x; 1.3303x over previous
import functools
import numpy as np
import jax
import jax.numpy as jnp
from jax import lax
from jax.experimental import pallas as pl
from jax.experimental.pallas import tpu as pltpu
from jax.experimental.pallas import tpu_sc as plsc

_N = 10000
_E = 320000
_D = 128
_A = 8
_R = 16
_H = 64
_NUM_NEIGHBORS = 32.0

_NW = 32
_CHUNK = 80
_NCH = 126
_EPW = _CHUNK * _NCH
_EPAD = _EPW * _NW
_NPAD = 10240
_NODES_PER_TILE = _NPAD // 16


def _node_fctp_body(x_ref, a_ref, wsc_ref, wl1_ref, nsc_ref, nf_ref):
    x = x_ref[...]
    a = a_ref[...]
    acc_sc = jnp.zeros_like(nsc_ref)
    acc_l1 = jnp.zeros_like(nf_ref)
    for j in range(_A):
        aj = a[:, j:j + 1]
        acc_sc += aj * jnp.dot(x, wsc_ref[j], preferred_element_type=jnp.float32)
        acc_l1 += aj * jnp.dot(x, wl1_ref[j], preferred_element_type=jnp.float32)
    scale = 1.0 / np.sqrt(_D * _A)
    nsc_ref[...] = acc_sc * scale
    nf_ref[...] = acc_l1 * scale


def _edge_mlp_body(emb_ref, ea_ref, w1_ref, w2_ref, ew_ref):
    h = jnp.dot(emb_ref[...], w1_ref[...], preferred_element_type=jnp.float32)
    h = jax.nn.silu(h * (1.0 / np.sqrt(_R)))
    w = jnp.dot(h, w2_ref[...], preferred_element_type=jnp.float32)
    ew_ref[...] = w * (1.0 / np.sqrt(_H)) * ea_ref[...]


def _sc_edge_body(nf_hbm, ew_hbm, idx_hbm, zeros_hbm, out_hbm,
                  ib0, ib1, ib2, rows0, rows1, ewv0, ewv1, agg,
                  is0, is1, is2, gs0, gs1, es0, es1, ss0, ss1):
    c = lax.axis_index("c")
    s = lax.axis_index("s")
    wid = c * 16 + s
    stripe = pl.ds(s * _NODES_PER_TILE, _NODES_PER_TILE)
    pltpu.sync_copy(zeros_hbm.at[stripe], agg.at[stripe])
    plsc.subcore_barrier()

    ibufs = (ib0, ib1, ib2)
    isem = (is0, is1, is2)
    rows = (rows0, rows1)
    ewv = (ewv0, ewv1)
    gs = (gs0, gs1)
    es = (es0, es1)
    ss = (ss0, ss1)

    def issue_ge(q, b, ib):
        pltpu.async_copy(nf_hbm.at[ibufs[ib].at[0]], rows[b], gs[b])
        pltpu.async_copy(ew_hbm.at[wid, q], ewv[b], es[b])

    pltpu.sync_copy(idx_hbm.at[wid, 0], ib0)
    pltpu.async_copy(idx_hbm.at[wid, 1], ib1, is1)
    issue_ge(0, 0, 0)

    @pl.loop(0, _NCH, step=6)
    def _six(k):
        for r in range(6):
            q = k + r
            b = r % 2
            nb = 1 - b
            ib_cur = r % 3
            ib_next = (r + 1) % 3
            ib_pf = (r + 2) % 3

            @pl.when(q + 1 < _NCH)
            def _pf():
                pltpu.make_async_copy(
                    idx_hbm.at[wid, q + 1], ibufs[ib_next], isem[ib_next]
                ).wait()

                @pl.when(q >= 1)
                def _w():
                    pltpu.make_async_copy(
                        rows[nb], agg.at[ibufs[ib_next].at[1]], ss[nb]).wait()
                issue_ge(q + 1, nb, ib_next)

            @pl.when(q + 2 < _NCH)
            def _pfi():
                pltpu.async_copy(idx_hbm.at[wid, q + 2], ibufs[ib_pf],
                                 isem[ib_pf])

            pltpu.make_async_copy(nf_hbm.at[ibufs[ib_cur].at[0]], rows[b],
                                  gs[b]).wait()
            pltpu.make_async_copy(ew_hbm.at[wid, q], ewv[b], es[b]).wait()

            @pl.loop(0, _CHUNK)
            def _row(e):
                for t in range(_D // 16):
                    sl = pl.ds(t * 16, 16)
                    rows[b][e, sl] = rows[b][e, sl] * ewv[b][e, sl]

            pltpu.async_copy(rows[b], agg.at[ibufs[ib_cur].at[1]], ss[b],
                             add=True)

    pltpu.make_async_copy(rows0, agg.at[ib0.at[1]], ss0).wait()
    pltpu.make_async_copy(rows1, agg.at[ib0.at[1]], ss1).wait()
    plsc.subcore_barrier()
    pltpu.sync_copy(agg.at[stripe], out_hbm.at[c, stripe])


def _sc_edge_pass(nf, ew, isrc_p, jdst_p):
    f32 = jnp.float32
    nf_p = jnp.pad(nf, ((0, _NPAD - _N), (0, 0)))
    zeros_nd = jnp.zeros((_NPAD, _D), f32)
    isrc_3d = isrc_p.reshape(_NW, _NCH, _CHUNK)
    jdst_3d = jdst_p.reshape(_NW, _NCH, _CHUNK)
    idx_4d = jnp.stack([isrc_3d, jdst_3d], axis=2)
    ew_4d = ew.reshape(_NW, _NCH, _CHUNK, _D)
    mesh = plsc.VectorSubcoreMesh(core_axis_name="c", subcore_axis_name="s",
                                  num_cores=2, num_subcores=16)
    return pl.kernel(
        _sc_edge_body,
        out_type=jax.ShapeDtypeStruct((2, _NPAD, _D), f32),
        mesh=mesh,
        scratch_types=[
            pltpu.VMEM((2, _CHUNK), jnp.int32),
            pltpu.VMEM((2, _CHUNK), jnp.int32),
            pltpu.VMEM((2, _CHUNK), jnp.int32),
            pltpu.VMEM((_CHUNK, _D), f32),
            pltpu.VMEM((_CHUNK, _D), f32),
            pltpu.VMEM((_CHUNK, _D), f32),
            pltpu.VMEM((_CHUNK, _D), f32),
            pltpu.VMEM_SHARED((_NPAD, _D), f32),
        ] + [pltpu.SemaphoreType.DMA] * 9,
    )(nf_p, ew_4d, idx_4d, zeros_nd)


def _final_body(p_ref, a_ref, nsc_ref, wl2_ref, wa_ref, out_ref):
    agg = (p_ref[0] + p_ref[1]) * (1.0 / np.sqrt(_NUM_NEIGHBORS))
    a = a_ref[...]
    acc = jnp.zeros_like(out_ref)
    for j in range(_A):
        acc += a[:, j:j + 1] * jnp.dot(agg, wl2_ref[j],
                                       preferred_element_type=jnp.float32)
    scale = 1.0 / np.sqrt(_D * _A)
    nco = acc * scale
    va = jnp.dot(agg, wa_ref[...], preferred_element_type=jnp.float32)
    alpha = jnp.sum(va * a, axis=1, keepdims=True) * scale
    out_ref[...] = nsc_ref[...] + alpha * nco


def kernel(x, node_attr, edge_index, edge_attr, edge_len_emb,
           W_sc, W_lin1, W_lin2, W_alpha, W_mlp1, W_mlp2):
    f32 = jnp.float32
    x = x.astype(f32)
    isrc = edge_index[0].astype(jnp.int32)
    jdst = edge_index[1].astype(jnp.int32)

    wsc_t = jnp.transpose(W_sc, (1, 0, 2))
    wl1_t = jnp.transpose(W_lin1, (1, 0, 2))
    wl2_t = jnp.transpose(W_lin2, (1, 0, 2))

    bn = 1000
    grid_n = _N // bn
    nsc, nf = pl.pallas_call(
        _node_fctp_body,
        grid=(grid_n,),
        in_specs=[
            pl.BlockSpec((bn, _D), lambda i: (i, 0)),
            pl.BlockSpec((bn, _A), lambda i: (i, 0)),
            pl.BlockSpec((_A, _D, _D), lambda i: (0, 0, 0)),
            pl.BlockSpec((_A, _D, _D), lambda i: (0, 0, 0)),
        ],
        out_specs=[
            pl.BlockSpec((bn, _D), lambda i: (i, 0)),
            pl.BlockSpec((bn, _D), lambda i: (i, 0)),
        ],
        out_shape=[
            jax.ShapeDtypeStruct((_N, _D), f32),
            jax.ShapeDtypeStruct((_N, _D), f32),
        ],
    )(x, node_attr, wsc_t, wl1_t)

    emb_p = jnp.pad(edge_len_emb.astype(f32), ((0, _EPAD - _E), (0, 0)))
    ea_p = jnp.pad(edge_attr.astype(f32), ((0, _EPAD - _E), (0, 0)))
    isrc_p = jnp.pad(isrc, (0, _EPAD - _E))
    jdst_p = jnp.pad(jdst, (0, _EPAD - _E))

    be = 5120
    grid_e = _EPAD // be
    ew = pl.pallas_call(
        _edge_mlp_body,
        grid=(grid_e,),
        in_specs=[
            pl.BlockSpec((be, _R), lambda i: (i, 0)),
            pl.BlockSpec((be, 1), lambda i: (i, 0)),
            pl.BlockSpec((_R, _H), lambda i: (0, 0)),
            pl.BlockSpec((_H, _D), lambda i: (0, 0)),
        ],
        out_specs=pl.BlockSpec((be, _D), lambda i: (i, 0)),
        out_shape=jax.ShapeDtypeStruct((_EPAD, _D), f32),
    )(emb_p, ea_p, W_mlp1.astype(f32), W_mlp2.astype(f32))

    partials = _sc_edge_pass(nf, ew, isrc_p, jdst_p)

    wa_mat = W_alpha.reshape(_D, _A).astype(f32)
    out = pl.pallas_call(
        _final_body,
        grid=(grid_n,),
        in_specs=[
            pl.BlockSpec((2, bn, _D), lambda i: (0, i, 0)),
            pl.BlockSpec((bn, _A), lambda i: (i, 0)),
            pl.BlockSpec((bn, _D), lambda i: (i, 0)),
            pl.BlockSpec((_A, _D, _D), lambda i: (0, 0, 0)),
            pl.BlockSpec((_D, _A), lambda i: (0, 0)),
        ],
        out_specs=pl.BlockSpec((bn, _D), lambda i: (i, 0)),
        out_shape=jax.ShapeDtypeStruct((_N, _D), f32),
    )(partials, node_attr, nsc, wl2_t, wa_mat)
    return out


# speedup vs baseline: 2.8100x; 1.2166x over previous
import functools
import numpy as np
import jax
import jax.numpy as jnp
from jax import lax
from jax.experimental import pallas as pl
from jax.experimental.pallas import tpu as pltpu
from jax.experimental.pallas import tpu_sc as plsc

_N = 10000
_E = 320000
_D = 128
_A = 8
_R = 16
_H = 64
_NUM_NEIGHBORS = 32.0

_NW = 32
_CHUNK = 80
_NCH = 126
_EPW = _CHUNK * _NCH
_EPAD = _EPW * _NW
_NPAD = 10240
_NODES_PER_TILE = _NPAD // 16

_PERM = np.concatenate([
    np.concatenate([32 * t + 2 * np.arange(16),
                    32 * t + 2 * np.arange(16) + 1])
    for t in range(_D // 32)
])


def _node_fctp_body(x_ref, a_ref, wsc_ref, wl1_ref, nsc_ref, nf_ref):
    x = x_ref[...]
    a = a_ref[...]
    acc_sc = jnp.zeros_like(nsc_ref)
    acc_l1 = jnp.zeros_like(nf_ref)
    for j in range(_A):
        aj = a[:, j:j + 1]
        acc_sc += aj * jnp.dot(x, wsc_ref[j], preferred_element_type=jnp.float32)
        acc_l1 += aj * jnp.dot(x, wl1_ref[j], preferred_element_type=jnp.float32)
    scale = 1.0 / np.sqrt(_D * _A)
    nsc_ref[...] = acc_sc * scale
    nf_ref[...] = acc_l1 * scale


def _edge_mlp_body(emb_ref, ea_ref, w1_ref, w2_ref, ew_ref):
    h = jnp.dot(emb_ref[...], w1_ref[...], preferred_element_type=jnp.float32)
    h = jax.nn.silu(h * (1.0 / np.sqrt(_R)))
    w = jnp.dot(h, w2_ref[...], preferred_element_type=jnp.float32)
    be = ew_ref.shape[0]
    row0 = pl.program_id(0) * be
    in_range = (row0 + jax.lax.broadcasted_iota(jnp.int32, (be, 1), 0)) < _E
    w = jnp.where(in_range, w * (1.0 / np.sqrt(_H)) * ea_ref[...], 0.0)
    ew_ref[...] = w


def _sc_edge_body(nf_hbm, ew_hbm, idx_hbm, out_hbm,
                  ib0, ib1, ib2, rows0, rows1, ewv0, ewv1, agg,
                  is0, is1, is2, gs0, gs1, es0, es1, ss0, ss1):
    c = lax.axis_index("c")
    s = lax.axis_index("s")
    wid = c * 16 + s

    zvec = jnp.zeros((16,), jnp.float32)

    @pl.loop(0, _CHUNK)
    def _z(e):
        for t in range(_D // 16):
            rows0[e, pl.ds(t * 16, 16)] = zvec

    for u in range(_NODES_PER_TILE // _CHUNK):
        pltpu.sync_copy(
            rows0, agg.at[pl.ds(s * _NODES_PER_TILE + u * _CHUNK, _CHUNK)])
    plsc.subcore_barrier()

    ibufs = (ib0, ib1, ib2)
    isem = (is0, is1, is2)
    rows = (rows0, rows1)
    ewv = (ewv0, ewv1)
    gs = (gs0, gs1)
    es = (es0, es1)
    ss = (ss0, ss1)

    def ew_slice(q):
        return ew_hbm.at[wid, q]

    def issue_ge(q, b, ib):
        pltpu.async_copy(nf_hbm.at[ibufs[ib].at[0]], rows[b], gs[b])
        pltpu.async_copy(ew_slice(q), ewv[b], es[b])

    pltpu.sync_copy(idx_hbm.at[wid, 0], ib0)
    pltpu.async_copy(idx_hbm.at[wid, 1], ib1, is1)
    issue_ge(0, 0, 0)

    @pl.loop(0, _NCH, step=6)
    def _six(k):
        for r in range(6):
            q = k + r
            b = r % 2
            nb = 1 - b
            ib_cur = r % 3
            ib_next = (r + 1) % 3
            ib_pf = (r + 2) % 3

            @pl.when(q + 1 < _NCH)
            def _pf():
                pltpu.make_async_copy(
                    idx_hbm.at[wid, q + 1], ibufs[ib_next], isem[ib_next]
                ).wait()

                @pl.when(q >= 1)
                def _w():
                    pltpu.make_async_copy(
                        rows[nb], agg.at[ibufs[ib_next].at[1]], ss[nb]).wait()
                issue_ge(q + 1, nb, ib_next)

            @pl.when(q + 2 < _NCH)
            def _pfi():
                pltpu.async_copy(idx_hbm.at[wid, q + 2], ibufs[ib_pf],
                                 isem[ib_pf])

            pltpu.make_async_copy(nf_hbm.at[ibufs[ib_cur].at[0]], rows[b],
                                  gs[b]).wait()
            pltpu.make_async_copy(ew_slice(q), ewv[b], es[b]).wait()

            @pl.loop(0, _CHUNK)
            def _row(e):
                for t in range(_D // 16):
                    sl = pl.ds(t * 16, 16)
                    rows[b][e, sl] = rows[b][e, sl] * ewv[b][e, sl]

            pltpu.async_copy(rows[b], agg.at[ibufs[ib_cur].at[1]], ss[b],
                             add=True)

    pltpu.make_async_copy(rows0, agg.at[ib0.at[1]], ss0).wait()
    pltpu.make_async_copy(rows1, agg.at[ib0.at[1]], ss1).wait()
    plsc.subcore_barrier()
    stripe = pl.ds(s * _NODES_PER_TILE, _NODES_PER_TILE)
    pltpu.sync_copy(agg.at[stripe], out_hbm.at[c, stripe])


def _sc_edge_pass(nf, ew, isrc_p, jdst_p):
    f32 = jnp.float32
    isrc_3d = isrc_p.reshape(_NW, _NCH, _CHUNK)
    jdst_3d = jdst_p.reshape(_NW, _NCH, _CHUNK)
    idx_4d = jnp.stack([isrc_3d, jdst_3d], axis=2)
    ew_4d = ew.reshape(_NW, _NCH, _CHUNK, _D)
    mesh = plsc.VectorSubcoreMesh(core_axis_name="c", subcore_axis_name="s",
                                  num_cores=2, num_subcores=16)
    return pl.kernel(
        _sc_edge_body,
        out_type=jax.ShapeDtypeStruct((2, _NPAD, _D), f32),
        mesh=mesh,
        scratch_types=[
            pltpu.VMEM((2, _CHUNK), jnp.int32),
            pltpu.VMEM((2, _CHUNK), jnp.int32),
            pltpu.VMEM((2, _CHUNK), jnp.int32),
            pltpu.VMEM((_CHUNK, _D), f32),
            pltpu.VMEM((_CHUNK, _D), f32),
            pltpu.VMEM((_CHUNK, _D), f32),
            pltpu.VMEM((_CHUNK, _D), f32),
            pltpu.VMEM_SHARED((_NPAD, _D), f32),
        ] + [pltpu.SemaphoreType.DMA] * 9,
    )(nf, ew_4d, idx_4d)


def _final_body(p_ref, a_ref, nsc_ref, wl2_ref, wa_ref, out_ref):
    agg = (p_ref[0] + p_ref[1]) * (1.0 / np.sqrt(_NUM_NEIGHBORS))
    a = a_ref[...]
    acc = jnp.zeros_like(out_ref)
    for j in range(_A):
        acc += a[:, j:j + 1] * jnp.dot(agg, wl2_ref[j],
                                       preferred_element_type=jnp.float32)
    scale = 1.0 / np.sqrt(_D * _A)
    nco = acc * scale
    va = jnp.dot(agg, wa_ref[...], preferred_element_type=jnp.float32)
    alpha = jnp.sum(va * a, axis=1, keepdims=True) * scale
    out_ref[...] = nsc_ref[...] + alpha * nco


def kernel(x, node_attr, edge_index, edge_attr, edge_len_emb,
           W_sc, W_lin1, W_lin2, W_alpha, W_mlp1, W_mlp2):
    f32 = jnp.float32
    x = x.astype(f32)
    isrc = edge_index[0].astype(jnp.int32)
    jdst = edge_index[1].astype(jnp.int32)

    wsc_t = jnp.transpose(W_sc, (1, 0, 2))
    wl1_t = jnp.transpose(W_lin1, (1, 0, 2))
    wl2_t = jnp.transpose(W_lin2, (1, 0, 2))

    bn = 1000
    grid_n = _N // bn
    nsc, nf = pl.pallas_call(
        _node_fctp_body,
        grid=(grid_n,),
        in_specs=[
            pl.BlockSpec((bn, _D), lambda i: (i, 0)),
            pl.BlockSpec((bn, _A), lambda i: (i, 0)),
            pl.BlockSpec((_A, _D, _D), lambda i: (0, 0, 0)),
            pl.BlockSpec((_A, _D, _D), lambda i: (0, 0, 0)),
        ],
        out_specs=[
            pl.BlockSpec((bn, _D), lambda i: (i, 0)),
            pl.BlockSpec((bn, _D), lambda i: (i, 0)),
        ],
        out_shape=[
            jax.ShapeDtypeStruct((_N, _D), f32),
            jax.ShapeDtypeStruct((_NPAD, _D), f32),
        ],
    )(x, node_attr, wsc_t, wl1_t)

    isrc_p = jnp.pad(isrc, (0, _EPAD - _E))
    jdst_p = jnp.pad(jdst, (0, _EPAD - _E))

    be = 5120
    grid_e = _EPAD // be
    ew = pl.pallas_call(
        _edge_mlp_body,
        grid=(grid_e,),
        in_specs=[
            pl.BlockSpec((be, _R), lambda i: (i, 0)),
            pl.BlockSpec((be, 1), lambda i: (i, 0)),
            pl.BlockSpec((_R, _H), lambda i: (0, 0)),
            pl.BlockSpec((_H, _D), lambda i: (0, 0)),
        ],
        out_specs=pl.BlockSpec((be, _D), lambda i: (i, 0)),
        out_shape=jax.ShapeDtypeStruct((_EPAD, _D), f32),
    )(edge_len_emb.astype(f32), edge_attr.astype(f32),
      W_mlp1.astype(f32), W_mlp2.astype(f32))

    partials = _sc_edge_pass(nf, ew, isrc_p, jdst_p)

    wa_mat = W_alpha.reshape(_D, _A).astype(f32)
    out = pl.pallas_call(
        _final_body,
        grid=(grid_n,),
        in_specs=[
            pl.BlockSpec((2, bn, _D), lambda i: (0, i, 0)),
            pl.BlockSpec((bn, _A), lambda i: (i, 0)),
            pl.BlockSpec((bn, _D), lambda i: (i, 0)),
            pl.BlockSpec((_A, _D, _D), lambda i: (0, 0, 0)),
            pl.BlockSpec((_D, _A), lambda i: (0, 0)),
        ],
        out_specs=pl.BlockSpec((bn, _D), lambda i: (i, 0)),
        out_shape=jax.ShapeDtypeStruct((_N, _D), f32),
    )(partials, node_attr, nsc, wl2_t, wa_mat)
    return out
